# baseline (speedup 1.0000x reference)
import functools

import jax
import jax.numpy as jnp
from jax import lax
from jax.experimental import pallas as pl
from jax.experimental.pallas import tpu as pltpu
from jax.experimental.pallas import tpu_sc as plsc


def _build_adjacency(src, dst, w, num_nodes):
    E = src.shape[0]
    info = plsc.get_sparse_core_info()
    nc, ns = info.num_cores, info.num_subcores
    nwork = nc * ns
    rows = num_nodes // nwork
    mesh = plsc.VectorSubcoreMesh(core_axis_name="c", subcore_axis_name="s")

    @functools.partial(
        pl.kernel,
        out_type=jax.ShapeDtypeStruct((num_nodes, num_nodes), jnp.float32),
        mesh=mesh,
        compiler_params=pltpu.CompilerParams(needs_layout_passes=False),
        scratch_types=[
            pltpu.VMEM((E,), jnp.int32),
            pltpu.VMEM((E,), jnp.int32),
            pltpu.VMEM((E,), jnp.float32),
            pltpu.VMEM((rows, num_nodes), jnp.float32),
        ],
    )
    def k(src_hbm, dst_hbm, w_hbm, out_hbm, src_v, dst_v, w_v, acc_v):
        wid = lax.axis_index("s") * nc + lax.axis_index("c")
        lo = wid * rows
        pltpu.sync_copy(src_hbm, src_v)
        pltpu.sync_copy(dst_hbm, dst_v)
        pltpu.sync_copy(w_hbm, w_v)

        zero16 = jnp.zeros((16,), jnp.float32)

        def zrow(r, c):
            def zcol(j, c2):
                acc_v[r, pl.ds(j * 16, 16)] = zero16
                return c2

            lax.fori_loop(0, num_nodes // 16, zcol, 0, unroll=8)
            return c

        lax.fori_loop(0, rows, zrow, 0)

        def body(i, carry):
            s16 = src_v[pl.ds(i * 16, 16)]
            d16 = dst_v[pl.ds(i * 16, 16)]
            w16 = w_v[pl.ds(i * 16, 16)]
            rel = s16 - lo
            m = (rel >= 0) & (rel < rows)
            relc = jnp.where(m, rel, 0)
            plsc.addupdate_scatter(acc_v, [relc, d16], w16, mask=m)
            return carry

        lax.fori_loop(0, E // 16, body, 0, unroll=4)
        pltpu.sync_copy(acc_v, out_hbm.at[pl.ds(lo, rows)])

    return k(src, dst, w)


def _gnn_layers(x3, at, e1, benc_col, wbig, bl_cols, num_layers, bb):
    nblk, _, f = x3.shape
    c = e1.shape[0]

    def body(x_ref, at_ref, e1_ref, benc_ref, wbig_ref, blc_ref, out_ref):
        xb = x_ref[0]
        h = jnp.dot(e1_ref[...], xb, preferred_element_type=jnp.float32)
        h = h + benc_ref[...]
        for l in range(num_layers):
            hn = jnp.dot(wbig_ref[l], h, preferred_element_type=jnp.float32)
            hn = hn + blc_ref[l]
            msg = jnp.dot(hn, at_ref[...], preferred_element_type=jnp.float32)
            h = jnp.maximum(h + msg, 0.0)
        out_ref[...] = h

    return pl.pallas_call(
        body,
        grid=(nblk,),
        in_specs=[
            pl.BlockSpec((1, bb, f), lambda p: (p, 0, 0)),
            pl.BlockSpec((f, f), lambda p: (0, 0)),
            pl.BlockSpec((c, bb), lambda p: (0, 0)),
            pl.BlockSpec((c, 1), lambda p: (0, 0)),
            pl.BlockSpec((num_layers, c, c), lambda p: (0, 0, 0)),
            pl.BlockSpec((num_layers, c, 1), lambda p: (0, 0, 0)),
        ],
        out_specs=pl.BlockSpec((c, f), lambda p: (p, 0)),
        out_shape=jax.ShapeDtypeStruct((nblk * c, f), jnp.float32),
    )(x3, at, e1, benc_col, wbig, bl_cols)


def _transpose_wc1(wc1r, fc):
    o, f, h = wc1r.shape

    def body(in_ref, out_ref):
        out_ref[...] = jnp.transpose(in_ref[...], (2, 1, 0))

    return pl.pallas_call(
        body,
        grid=(f // fc,),
        in_specs=[pl.BlockSpec((o, fc, h), lambda p: (0, p, 0))],
        out_specs=pl.BlockSpec((h, fc, o), lambda p: (0, p, 0)),
        out_shape=jax.ShapeDtypeStruct((h, f, o), jnp.float32),
    )(wc1r)


def _classifier(h3, wc13, bc1_row, wc2t, bc2_row, hc):
    b, hh, f = h3.shape
    out = wc2t.shape[1]

    def body(h_ref, w_ref, bc1_ref, wc2_ref, bc2_ref, out_ref, acc):
        p = pl.program_id(0)

        @pl.when(p == 0)
        def _():
            acc[...] = jnp.zeros_like(acc)

        a = acc[...]
        for i in range(hc):
            a = a + jnp.dot(h_ref[:, i, :], w_ref[i],
                            preferred_element_type=jnp.float32)
        acc[...] = a

        @pl.when(p == pl.num_programs(0) - 1)
        def _():
            z = jnp.maximum(a + bc1_ref[...], 0.0)
            out_ref[...] = (
                jnp.dot(z, wc2_ref[...], preferred_element_type=jnp.float32)
                + bc2_ref[...]
            )

    return pl.pallas_call(
        body,
        grid=(hh // hc,),
        in_specs=[
            pl.BlockSpec((b, hc, f), lambda p: (0, p, 0)),
            pl.BlockSpec((hc, f, hh), lambda p: (p, 0, 0)),
            pl.BlockSpec((1, hh), lambda p: (0, 0)),
            pl.BlockSpec((hh, out), lambda p: (0, 0)),
            pl.BlockSpec((1, out), lambda p: (0, 0)),
        ],
        out_specs=pl.BlockSpec((b, out), lambda p: (0, 0)),
        out_shape=jax.ShapeDtypeStruct((b, out), jnp.float32),
        scratch_shapes=[pltpu.VMEM((b, hh), jnp.float32)],
    )(h3, wc13, bc1_row, wc2t, bc2_row)


def kernel(x, edge_index, edge_attr, w_enc, b_enc, Wls, bls, Wc1, bc1, Wc2, bc2):
    B, F = x.shape
    H = w_enc.shape[0]
    L = Wls.shape[0]
    OUT = Wc2.shape[0]
    BB = 4

    src = edge_index[0]
    dst = edge_index[1]
    w = edge_attr[:, 0]

    at = _build_adjacency(src, dst, w, F)

    eye = jnp.eye(BB, dtype=jnp.float32)
    e1 = jnp.kron(eye, w_enc)
    benc_col = jnp.tile(b_enc, BB)[:, None]
    wbig = jnp.stack([jnp.kron(eye, Wls[l]) for l in range(L)])
    bl_cols = jnp.tile(bls, (1, BB))[:, :, None]

    x3 = x.reshape(B // BB, BB, F)
    ht = _gnn_layers(x3, at, e1, benc_col, wbig, bl_cols, L, BB)

    wc13 = _transpose_wc1(Wc1.reshape(H, F, H), fc=128)
    h3 = ht.reshape(B, H, F)
    logits = _classifier(h3, wc13, bc1[None, :], Wc2.T, bc2[None, :], hc=8)
    return logits


# speedup vs baseline: 1.1963x; 1.1963x over previous
import functools

import jax
import jax.numpy as jnp
from jax import lax
from jax.experimental import pallas as pl
from jax.experimental.pallas import tpu as pltpu
from jax.experimental.pallas import tpu_sc as plsc


def _build_adjacency(edge_index, w, num_nodes):
    E = edge_index.shape[1]
    info = plsc.get_sparse_core_info()
    nc, ns = info.num_cores, info.num_subcores
    nwork = nc * ns
    rows = num_nodes // nwork
    mesh = plsc.VectorSubcoreMesh(core_axis_name="c", subcore_axis_name="s")

    @functools.partial(
        pl.kernel,
        out_type=jax.ShapeDtypeStruct((num_nodes, num_nodes), jnp.float32),
        mesh=mesh,
        compiler_params=pltpu.CompilerParams(needs_layout_passes=False),
        scratch_types=[
            pltpu.VMEM((E,), jnp.int32),
            pltpu.VMEM((E,), jnp.int32),
            pltpu.VMEM((E,), jnp.float32),
            pltpu.VMEM((rows, num_nodes), jnp.float32),
        ],
    )
    def k(ei_hbm, w_hbm, out_hbm, src_v, dst_v, w_v, acc_v):
        wid = lax.axis_index("s") * nc + lax.axis_index("c")
        lo = wid * rows
        pltpu.sync_copy(ei_hbm.at[0], src_v)
        pltpu.sync_copy(ei_hbm.at[1], dst_v)
        pltpu.sync_copy(w_hbm, w_v)

        zero16 = jnp.zeros((16,), jnp.float32)

        def zrow(r, c):
            def zcol(j, c2):
                acc_v[r, pl.ds(j * 16, 16)] = zero16
                return c2

            lax.fori_loop(0, num_nodes // 16, zcol, 0, unroll=8)
            return c

        lax.fori_loop(0, rows, zrow, 0)

        def body(i, carry):
            s16 = src_v[pl.ds(i * 16, 16)]
            d16 = dst_v[pl.ds(i * 16, 16)]
            w16 = w_v[pl.ds(i * 16, 16)]
            rel = s16 - lo
            m = (rel >= 0) & (rel < rows)
            relc = jnp.where(m, rel, 0)
            plsc.addupdate_scatter(acc_v, [relc, d16], w16, mask=m)
            return carry

        lax.fori_loop(0, E // 16, body, 0, unroll=4)
        pltpu.sync_copy(acc_v, out_hbm.at[pl.ds(lo, rows)])

    return k(edge_index, w)


def _gnn_layers(x3, at, e1, benc_col, wbig, bl_cols, num_layers, bb):
    nblk, _, f = x3.shape
    c = e1.shape[0]

    def body(x_ref, at_ref, e1_ref, benc_ref, wbig_ref, blc_ref, out_ref):
        xb = x_ref[0]
        h = jnp.dot(e1_ref[...], xb, preferred_element_type=jnp.float32)
        h = h + benc_ref[...]
        for l in range(num_layers):
            hn = jnp.dot(wbig_ref[l], h, preferred_element_type=jnp.float32)
            hn = hn + blc_ref[l]
            msg = jnp.dot(hn, at_ref[...], preferred_element_type=jnp.float32)
            h = jnp.maximum(h + msg, 0.0)
        out_ref[...] = h

    return pl.pallas_call(
        body,
        grid=(nblk,),
        in_specs=[
            pl.BlockSpec((1, bb, f), lambda p: (p, 0, 0)),
            pl.BlockSpec((f, f), lambda p: (0, 0)),
            pl.BlockSpec((c, bb), lambda p: (0, 0)),
            pl.BlockSpec((c, 1), lambda p: (0, 0)),
            pl.BlockSpec((num_layers, c, c), lambda p: (0, 0, 0)),
            pl.BlockSpec((num_layers, c, 1), lambda p: (0, 0, 0)),
        ],
        out_specs=pl.BlockSpec((c, f), lambda p: (p, 0)),
        out_shape=jax.ShapeDtypeStruct((nblk * c, f), jnp.float32),
    )(x3, at, e1, benc_col, wbig, bl_cols)


def _transpose_wc1(wc1r, fc):
    o, f, h = wc1r.shape

    def body(in_ref, out_ref):
        out_ref[...] = jnp.transpose(in_ref[...], (2, 1, 0))

    return pl.pallas_call(
        body,
        grid=(f // fc,),
        in_specs=[pl.BlockSpec((o, fc, h), lambda p: (0, p, 0))],
        out_specs=pl.BlockSpec((h, fc, o), lambda p: (0, p, 0)),
        out_shape=jax.ShapeDtypeStruct((h, f, o), jnp.float32),
    )(wc1r)


def _classifier(h3, wc13, bc1_row, wc2t, bc2_row, hc):
    b, hh, f = h3.shape
    out = wc2t.shape[1]

    def body(h_ref, w_ref, bc1_ref, wc2_ref, bc2_ref, out_ref, acc):
        p = pl.program_id(0)

        @pl.when(p == 0)
        def _():
            acc[...] = jnp.zeros_like(acc)

        a = acc[...]
        for i in range(hc):
            a = a + jnp.dot(h_ref[:, i, :], w_ref[i],
                            preferred_element_type=jnp.float32)
        acc[...] = a

        @pl.when(p == pl.num_programs(0) - 1)
        def _():
            z = jnp.maximum(a + bc1_ref[...], 0.0)
            out_ref[...] = (
                jnp.dot(z, wc2_ref[...], preferred_element_type=jnp.float32)
                + bc2_ref[...]
            )

    return pl.pallas_call(
        body,
        grid=(hh // hc,),
        in_specs=[
            pl.BlockSpec((b, hc, f), lambda p: (0, p, 0)),
            pl.BlockSpec((hc, f, hh), lambda p: (p, 0, 0)),
            pl.BlockSpec((1, hh), lambda p: (0, 0)),
            pl.BlockSpec((hh, out), lambda p: (0, 0)),
            pl.BlockSpec((1, out), lambda p: (0, 0)),
        ],
        out_specs=pl.BlockSpec((b, out), lambda p: (0, 0)),
        out_shape=jax.ShapeDtypeStruct((b, out), jnp.float32),
        scratch_shapes=[pltpu.VMEM((b, hh), jnp.float32)],
    )(h3, wc13, bc1_row, wc2t, bc2_row)


def kernel(x, edge_index, edge_attr, w_enc, b_enc, Wls, bls, Wc1, bc1, Wc2, bc2):
    B, F = x.shape
    H = w_enc.shape[0]
    L = Wls.shape[0]
    OUT = Wc2.shape[0]
    BB = 4

    w = edge_attr.reshape(-1)

    at = _build_adjacency(edge_index, w, F)

    eye = jnp.eye(BB, dtype=jnp.float32)
    e1 = jnp.kron(eye, w_enc)
    benc_col = jnp.tile(b_enc, BB)[:, None]
    wbig = jnp.stack([jnp.kron(eye, Wls[l]) for l in range(L)])
    bl_cols = jnp.tile(bls, (1, BB))[:, :, None]

    x3 = x.reshape(B // BB, BB, F)
    ht = _gnn_layers(x3, at, e1, benc_col, wbig, bl_cols, L, BB)

    wc13 = Wc1.reshape(H, F, H).transpose(2, 1, 0)
    h3 = ht.reshape(B, H, F)
    logits = _classifier(h3, wc13, bc1[None, :], Wc2.T, bc2[None, :], hc=8)
    return logits
